# baseline (speedup 1.0000x reference)
import functools

import jax
import jax.numpy as jnp
import numpy as np
from jax import lax
from jax.experimental import pallas as pl
from jax.experimental.pallas import tpu as pltpu
from jax.experimental.pallas import tpu_sc as plsc

_N = 10000
_D = 128
_E = 320000
_NPAD = 10240
_NC = 2
_NS = 16
_NW = _NC * _NS
_EPW = _E // _NW
_CH = 96
_NCHUNK = 107
_EPW_PAD = _NCHUNK * _CH
_NBUF = 3
_KI = 6
_ZC = 64
_RPT = _NPAD // _NS
_SBUF = 2
_SLOPE = 0.2

_PSRC = np.empty((128,), np.int64)
for _q in range(4):
    for _j in range(16):
        for _r in range(2):
            _PSRC[32 * _q + 2 * _j + _r] = 32 * _q + 16 * _r + _j
_PMAT = np.zeros((128, 128), np.float32)
_PMAT[_PSRC, np.arange(128)] = 1.0


def _emit_transform(x, w_ref, p_ref, as_ref, ad_ref, hb_ref, acs_ref,
                    acd_ref, bnd_ref):
    w = w_ref[...]
    wp = jnp.dot(w, p_ref[...], preferred_element_type=jnp.float32)
    hp = jnp.dot(x, wp, preferred_element_type=jnp.float32)
    hb_ref[...] = hp.astype(jnp.bfloat16)
    was = lax.dot_general(w, as_ref[...], (((1,), (1,)), ((), ())),
                          preferred_element_type=jnp.float32)
    wad = lax.dot_general(w, ad_ref[...], (((1,), (1,)), ((), ())),
                          preferred_element_type=jnp.float32)
    acs = jnp.dot(x, was, preferred_element_type=jnp.float32)
    acd = jnp.dot(x, wad, preferred_element_type=jnp.float32)
    acs_ref[...] = acs
    acd_ref[...] = acd
    bm = jnp.max(acs) + jnp.max(acd)
    bnd_ref[...] = jnp.full((1, 128), jnp.where(bm > 0, bm, _SLOPE * bm),
                            jnp.float32)


def _tc_transform_body(x_ref, w_ref, p_ref, as_ref, ad_ref,
                       hb_ref, acs_ref, acd_ref, bnd_ref):
    _emit_transform(x_ref[...], w_ref, p_ref, as_ref, ad_ref,
                    hb_ref, acs_ref, acd_ref, bnd_ref)


_tc_transform = pl.pallas_call(
    _tc_transform_body,
    out_shape=[
        jax.ShapeDtypeStruct((_N, _D), jnp.bfloat16),
        jax.ShapeDtypeStruct((_N, 1), jnp.float32),
        jax.ShapeDtypeStruct((_N, 1), jnp.float32),
        jax.ShapeDtypeStruct((1, 128), jnp.float32),
    ],
)


def _emit_combine(acc_ref, dv_ref, b_ref):
    o = (acc_ref[0, 0:_N, :] + acc_ref[1, 0:_N, :]) / dv_ref[0:_N, :]
    o = o + b_ref[...]
    return jnp.where(o > 0, o, jnp.exp(jnp.minimum(o, 0.0)) - 1.0)


def _tc_combine_transform_body(acc_ref, dv_ref, b_ref, w_ref, p_ref, as_ref,
                               ad_ref, x_ref, hb_ref, acs_ref, acd_ref,
                               bnd_ref):
    x = _emit_combine(acc_ref, dv_ref, b_ref)
    x_ref[...] = x
    _emit_transform(x, w_ref, p_ref, as_ref, ad_ref, hb_ref, acs_ref,
                    acd_ref, bnd_ref)


_tc_combine_transform = pl.pallas_call(
    _tc_combine_transform_body,
    out_shape=[
        jax.ShapeDtypeStruct((_N, _D), jnp.float32),
        jax.ShapeDtypeStruct((_N, _D), jnp.bfloat16),
        jax.ShapeDtypeStruct((_N, 1), jnp.float32),
        jax.ShapeDtypeStruct((_N, 1), jnp.float32),
        jax.ShapeDtypeStruct((1, 128), jnp.float32),
    ],
)


def _tc_combine_body(acc_ref, dv_ref, b_ref, x_ref):
    x_ref[...] = _emit_combine(acc_ref, dv_ref, b_ref)


_tc_combine = pl.pallas_call(
    _tc_combine_body,
    out_shape=jax.ShapeDtypeStruct((_N, _D), jnp.float32),
)


def _sc_body(h_hbm, src_hbm, dst_hbm, as_hbm, ad_hbm, bnd_hbm,
             acc_out, dacc_out,
             srcc, dstc, exv, asg, adg, btab, rowb, rowf, z1d,
             acc_sh, dacc_sh, gsem, ssem, isem):
    c = lax.axis_index("c")
    s = lax.axis_index("s")
    wid = s * _NC + c
    ebase = wid * _EPW_PAD
    zero16 = jnp.zeros((16,), jnp.float32)

    @pl.loop(0, _ZC)
    def _(i):
        for k in range(_D // 16):
            rowf[0, i, pl.ds(k * 16, 16)] = zero16

    @pl.loop(0, _RPT // 16)
    def _(i):
        z1d[pl.ds(i * 16, 16)] = zero16

    zsrc = rowf.at[0, pl.ds(0, _ZC), :]

    @pl.loop(0, _RPT // _ZC)
    def _(i):
        pltpu.sync_copy(zsrc, acc_sh.at[pl.ds(s * _RPT + i * _ZC, _ZC), :])

    pltpu.sync_copy(z1d, dacc_sh.at[pl.ds(s * _RPT, _RPT)])
    plsc.subcore_barrier()

    pltpu.sync_copy(bnd_hbm, btab)
    bound = btab[...][0]

    lane = lax.iota(jnp.int32, 16)

    def start_i(j, bi):
        off = ebase + j * _CH
        pltpu.async_copy(src_hbm.at[pl.ds(off, _CH)], srcc.at[bi],
                         isem.at[bi])
        pltpu.async_copy(dst_hbm.at[pl.ds(off, _CH)], dstc.at[bi],
                         isem.at[bi])

    def wait_i(j, bi):
        off = ebase + j * _CH
        pltpu.make_async_copy(src_hbm.at[pl.ds(off, _CH)], srcc.at[bi],
                              isem.at[bi]).wait()
        pltpu.make_async_copy(dst_hbm.at[pl.ds(off, _CH)], dstc.at[bi],
                              isem.at[bi]).wait()

    def start_g(j, b, bi):
        pltpu.async_copy(h_hbm.at[srcc.at[bi]], rowb.at[b], gsem.at[b])
        pltpu.async_copy(as_hbm.at[srcc.at[bi]], asg.at[b], gsem.at[b])
        pltpu.async_copy(ad_hbm.at[dstc.at[bi]], adg.at[b], gsem.at[b])

    def wait_g(j, b, bi):
        pltpu.make_async_copy(h_hbm.at[srcc.at[bi]], rowb.at[b],
                              gsem.at[b]).wait()
        pltpu.make_async_copy(as_hbm.at[srcc.at[bi]], asg.at[b],
                              gsem.at[b]).wait()
        pltpu.make_async_copy(ad_hbm.at[dstc.at[bi]], adg.at[b],
                              gsem.at[b]).wait()

    def start_s(j, sb, bi):
        pltpu.async_copy(rowf.at[sb], acc_sh.at[dstc.at[bi]], ssem.at[sb],
                         add=True)
        pltpu.async_copy(exv.at[sb], dacc_sh.at[dstc.at[bi]], ssem.at[sb],
                         add=True)

    def wait_s(j, sb, bi):
        pltpu.make_async_copy(rowf.at[sb], acc_sh.at[dstc.at[bi]],
                              ssem.at[sb]).wait()
        pltpu.make_async_copy(exv.at[sb], dacc_sh.at[dstc.at[bi]],
                              ssem.at[sb]).wait()

    def compute(j, b, sb):
        base = j * _CH
        for q in range(_CH // 16):
            sl16 = pl.ds(q * 16, 16)
            av = asg[b, sl16] + adg[b, sl16]
            e = jnp.where(av > 0, av, _SLOPE * av)
            ex = jnp.exp(e - bound)
            pos = base + q * 16 + lane
            ex = jnp.where(pos < _EPW, ex, 0.0)
            exv[sb, sl16] = ex

        @pl.loop(0, _CH // 16)
        def _(g):
            exw = exv[sb, pl.ds(g * 16, 16)]
            for l in range(16):
                r = g * 16 + l
                exs = exw[l]
                for k in range(_D // 32):
                    w16 = rowb[b, r, pl.ds(16 * k, 16)]
                    w32 = plsc.bitcast(w16, jnp.bfloat16)
                    ae, bo = plsc.unpack(
                        w32, format=plsc.PackFormat.INTERLEAVED)
                    rowf[sb, r, pl.ds(32 * k, 16)] = ae * exs
                    rowf[sb, r, pl.ds(32 * k + 16, 16)] = bo * exs

    def step(j, b):
        bi = j % _KI
        sb = j % _SBUF
        wait_g(j, b, bi)
        compute(j, b, sb)
        start_s(j, sb, bi)

        @pl.when(j >= 1)
        def _():
            wait_s(j - 1, (j - 1) % _SBUF, (j - 1) % _KI)

        @pl.when(j + 2 < _NCHUNK)
        def _():
            wait_i(j + 2, (j + 2) % _KI)
            start_g(j + 2, (b + 2) % _NBUF, (j + 2) % _KI)

        @pl.when(j + 4 < _NCHUNK)
        def _():
            start_i(j + 4, (j + 4) % _KI)

    for j in range(4):
        start_i(j, j)
    wait_i(0, 0)
    start_g(0, 0, 0)
    wait_i(1, 1)
    start_g(1, 1, 1)

    nmain = (_NCHUNK // _NBUF) * _NBUF

    @pl.loop(0, _NCHUNK // _NBUF)
    def _(t):
        for b in range(_NBUF):
            step(t * _NBUF + b, b)

    for j in range(nmain, _NCHUNK):
        step(j, j % _NBUF)
    wait_s(_NCHUNK - 1, (_NCHUNK - 1) % _SBUF, (_NCHUNK - 1) % _KI)
    plsc.subcore_barrier()

    @pl.loop(0, _RPT // _ZC)
    def _(i):
        row0 = s * _RPT + i * _ZC
        pltpu.sync_copy(acc_sh.at[pl.ds(row0, _ZC), :], zsrc)
        pltpu.sync_copy(zsrc, acc_out.at[c, pl.ds(row0, _ZC), :])

    pltpu.sync_copy(dacc_sh.at[pl.ds(s * _RPT, _RPT)], z1d)
    pltpu.sync_copy(z1d, dacc_out.at[c, pl.ds(s * _RPT, _RPT)])


_sc_edge = pl.kernel(
    _sc_body,
    out_type=[
        jax.ShapeDtypeStruct((_NC, _NPAD, _D), jnp.float32),
        jax.ShapeDtypeStruct((_NC, _NPAD), jnp.float32),
    ],
    mesh=plsc.VectorSubcoreMesh(core_axis_name="c", subcore_axis_name="s",
                                num_cores=_NC, num_subcores=_NS),
    compiler_params=pltpu.CompilerParams(needs_layout_passes=False,
                                         use_tc_tiling_on_sc=False),
    scratch_types=[
        pltpu.VMEM((_KI, _CH), jnp.int32),
        pltpu.VMEM((_KI, _CH), jnp.int32),
        pltpu.VMEM((_SBUF, _CH), jnp.float32),
        pltpu.VMEM((_NBUF, _CH), jnp.float32),
        pltpu.VMEM((_NBUF, _CH), jnp.float32),
        pltpu.VMEM((16,), jnp.float32),
        pltpu.VMEM((_NBUF, _CH, _D // 2), jnp.int32),
        pltpu.VMEM((_SBUF, _CH, _D), jnp.float32),
        pltpu.VMEM((_RPT,), jnp.float32),
        pltpu.VMEM_SHARED((_NPAD, _D), jnp.float32),
        pltpu.VMEM_SHARED((_NPAD,), jnp.float32),
        pltpu.SemaphoreType.DMA((_NBUF,)),
        pltpu.SemaphoreType.DMA((_SBUF,)),
        pltpu.SemaphoreType.DMA((_KI,)),
    ],
)


def kernel(x0, edge_index, W0, att_src0, att_dst0, b0,
           W1, att_src1, att_dst1, b1):
    src = edge_index[0].astype(jnp.int32)
    dst = edge_index[1].astype(jnp.int32)
    npad_e = _EPW_PAD - _EPW
    pad_nodes = (jnp.arange(_NW * npad_e, dtype=jnp.int32) % _N).reshape(
        _NW, npad_e)
    srcp = jnp.concatenate([src.reshape(_NW, _EPW), pad_nodes],
                           axis=1).reshape(_NW * _EPW_PAD)
    dstp = jnp.concatenate([dst.reshape(_NW, _EPW), pad_nodes],
                           axis=1).reshape(_NW * _EPW_PAD)

    pmat = jnp.asarray(_PMAT)

    def _as_i32_rows(hb):
        return lax.bitcast_convert_type(hb.reshape(_N, _D // 2, 2),
                                        jnp.int32)

    h, acs, acd, bnd = _tc_transform(x0, W0, pmat, att_src0, att_dst0)
    acc, dacc = _sc_edge(_as_i32_rows(h), srcp, dstp,
                         acs.reshape(_N), acd.reshape(_N),
                         bnd.reshape(128)[:16])
    dv = (dacc[0] + dacc[1] + 1e-16).reshape(_NPAD, 1)
    x1, h1, acs1, acd1, bnd1 = _tc_combine_transform(
        acc, dv, b0.reshape(1, _D), W1, pmat, att_src1, att_dst1)
    acc1, dacc1 = _sc_edge(_as_i32_rows(h1), srcp, dstp, acs1.reshape(_N),
                           acd1.reshape(_N), bnd1.reshape(128)[:16])
    dv1 = (dacc1[0] + dacc1[1] + 1e-16).reshape(_NPAD, 1)
    x2 = _tc_combine(acc1, dv1, b1.reshape(1, _D))
    return (x0, x1, x2)


# speedup vs baseline: 1.0024x; 1.0024x over previous
import functools

import jax
import jax.numpy as jnp
import numpy as np
from jax import lax
from jax.experimental import pallas as pl
from jax.experimental.pallas import tpu as pltpu
from jax.experimental.pallas import tpu_sc as plsc

_N = 10000
_D = 128
_E = 320000
_NPAD = 10240
_NC = 2
_NS = 16
_NW = _NC * _NS
_EPW = _E // _NW
_CH = 96
_NCHUNK = 107
_EPW_PAD = _NCHUNK * _CH
_NBUF = 3
_KI = 6
_ZC = 64
_RPT = _NPAD // _NS
_SBUF = 2
_SLOPE = 0.2

_PSRC = np.empty((128,), np.int64)
for _q in range(4):
    for _j in range(16):
        for _r in range(2):
            _PSRC[32 * _q + 2 * _j + _r] = 32 * _q + 16 * _r + _j
_PMAT = np.zeros((128, 128), np.float32)
_PMAT[_PSRC, np.arange(128)] = 1.0


def _emit_transform(x, w_ref, p_ref, as_ref, ad_ref, hb_ref, acs_ref,
                    acd_ref, bnd_ref):
    w = w_ref[...]
    wp = jnp.dot(w, p_ref[...], preferred_element_type=jnp.float32)
    hp = jnp.dot(x, wp, preferred_element_type=jnp.float32)
    hb_ref[...] = hp.astype(jnp.bfloat16)
    was = lax.dot_general(w, as_ref[...], (((1,), (1,)), ((), ())),
                          preferred_element_type=jnp.float32)
    wad = lax.dot_general(w, ad_ref[...], (((1,), (1,)), ((), ())),
                          preferred_element_type=jnp.float32)
    acs = jnp.dot(x, was, preferred_element_type=jnp.float32)
    acd = jnp.dot(x, wad, preferred_element_type=jnp.float32)
    acs_ref[...] = acs
    acd_ref[...] = acd
    bm = jnp.max(acs) + jnp.max(acd)
    bnd_ref[...] = jnp.full((1, 128), jnp.where(bm > 0, bm, _SLOPE * bm),
                            jnp.float32)


def _tc_transform_body(x_ref, w_ref, p_ref, as_ref, ad_ref,
                       hb_ref, acs_ref, acd_ref, bnd_ref):
    _emit_transform(x_ref[...], w_ref, p_ref, as_ref, ad_ref,
                    hb_ref, acs_ref, acd_ref, bnd_ref)


_tc_transform = pl.pallas_call(
    _tc_transform_body,
    out_shape=[
        jax.ShapeDtypeStruct((_N, _D), jnp.bfloat16),
        jax.ShapeDtypeStruct((_N, 1), jnp.float32),
        jax.ShapeDtypeStruct((_N, 1), jnp.float32),
        jax.ShapeDtypeStruct((1, 128), jnp.float32),
    ],
)


def _emit_combine(acc_ref, dv_ref, b_ref):
    o = (acc_ref[0, 0:_N, :] + acc_ref[1, 0:_N, :]) / dv_ref[0:_N, :]
    o = o + b_ref[...]
    return jnp.where(o > 0, o, jnp.exp(jnp.minimum(o, 0.0)) - 1.0)


def _tc_combine_transform_body(acc_ref, dv_ref, b_ref, w_ref, p_ref, as_ref,
                               ad_ref, x_ref, hb_ref, acs_ref, acd_ref,
                               bnd_ref):
    x = _emit_combine(acc_ref, dv_ref, b_ref)
    x_ref[...] = x
    _emit_transform(x, w_ref, p_ref, as_ref, ad_ref, hb_ref, acs_ref,
                    acd_ref, bnd_ref)


_tc_combine_transform = pl.pallas_call(
    _tc_combine_transform_body,
    out_shape=[
        jax.ShapeDtypeStruct((_N, _D), jnp.float32),
        jax.ShapeDtypeStruct((_N, _D), jnp.bfloat16),
        jax.ShapeDtypeStruct((_N, 1), jnp.float32),
        jax.ShapeDtypeStruct((_N, 1), jnp.float32),
        jax.ShapeDtypeStruct((1, 128), jnp.float32),
    ],
)


def _tc_combine_body(acc_ref, dv_ref, b_ref, x_ref):
    x_ref[...] = _emit_combine(acc_ref, dv_ref, b_ref)


_tc_combine = pl.pallas_call(
    _tc_combine_body,
    out_shape=jax.ShapeDtypeStruct((_N, _D), jnp.float32),
)


def _sc_body(h_hbm, src_hbm, dst_hbm, as_hbm, ad_hbm, bnd_hbm,
             acc_out, dacc_out,
             srcc, dstc, exv, asg, adg, btab, rowb, rowf, z1d,
             acc_sh, dacc_sh, gsem, ssem, isem):
    c = lax.axis_index("c")
    s = lax.axis_index("s")
    wid = s * _NC + c
    ebase = wid * _EPW_PAD
    zero16 = jnp.zeros((16,), jnp.float32)

    @pl.loop(0, _ZC)
    def _(i):
        for k in range(_D // 16):
            rowf[0, i, pl.ds(k * 16, 16)] = zero16

    @pl.loop(0, _RPT // 16)
    def _(i):
        z1d[pl.ds(i * 16, 16)] = zero16

    zsrc = rowf.at[0, pl.ds(0, _ZC), :]

    @pl.loop(0, _RPT // _ZC)
    def _(i):
        pltpu.sync_copy(zsrc, acc_sh.at[pl.ds(s * _RPT + i * _ZC, _ZC), :])

    pltpu.sync_copy(z1d, dacc_sh.at[pl.ds(s * _RPT, _RPT)])
    plsc.subcore_barrier()

    pltpu.sync_copy(bnd_hbm, btab)
    bound = btab[...][0]

    lane = lax.iota(jnp.int32, 16)

    def start_i(j, bi):
        off = ebase + j * _CH
        pltpu.async_copy(src_hbm.at[pl.ds(off, _CH)], srcc.at[bi],
                         isem.at[bi])
        pltpu.async_copy(dst_hbm.at[pl.ds(off, _CH)], dstc.at[bi],
                         isem.at[bi])

    def wait_i(j, bi):
        off = ebase + j * _CH
        pltpu.make_async_copy(src_hbm.at[pl.ds(off, _CH)], srcc.at[bi],
                              isem.at[bi]).wait()
        pltpu.make_async_copy(dst_hbm.at[pl.ds(off, _CH)], dstc.at[bi],
                              isem.at[bi]).wait()

    def start_g(j, b, bi):
        pltpu.async_copy(h_hbm.at[srcc.at[bi]], rowb.at[b], gsem.at[b])
        pltpu.async_copy(as_hbm.at[srcc.at[bi]], asg.at[b], gsem.at[b])
        pltpu.async_copy(ad_hbm.at[dstc.at[bi]], adg.at[b], gsem.at[b])

    def wait_g(j, b, bi):
        pltpu.make_async_copy(h_hbm.at[srcc.at[bi]], rowb.at[b],
                              gsem.at[b]).wait()
        pltpu.make_async_copy(as_hbm.at[srcc.at[bi]], asg.at[b],
                              gsem.at[b]).wait()
        pltpu.make_async_copy(ad_hbm.at[dstc.at[bi]], adg.at[b],
                              gsem.at[b]).wait()

    def start_s(j, sb, bi):
        pltpu.async_copy(rowf.at[sb], acc_sh.at[dstc.at[bi]], ssem.at[sb],
                         add=True)
        pltpu.async_copy(exv.at[sb], dacc_sh.at[dstc.at[bi]], ssem.at[sb],
                         add=True)

    def wait_s(j, sb, bi):
        pltpu.make_async_copy(rowf.at[sb], acc_sh.at[dstc.at[bi]],
                              ssem.at[sb]).wait()
        pltpu.make_async_copy(exv.at[sb], dacc_sh.at[dstc.at[bi]],
                              ssem.at[sb]).wait()

    def compute(j, b, sb):
        base = j * _CH
        for q in range(_CH // 16):
            sl16 = pl.ds(q * 16, 16)
            av = asg[b, sl16] + adg[b, sl16]
            e = jnp.where(av > 0, av, _SLOPE * av)
            ex = jnp.exp(e - bound)
            pos = base + q * 16 + lane
            ex = jnp.where(pos < _EPW, ex, 0.0)
            exv[sb, sl16] = ex

        @pl.loop(0, _CH // 16)
        def _(g):
            exw = exv[sb, pl.ds(g * 16, 16)]
            for l in range(16):
                r = g * 16 + l
                exs = exw[l]
                for k in range(_D // 32):
                    w16 = rowb[b, r, pl.ds(16 * k, 16)]
                    ae = plsc.bitcast(w16 << 16, jnp.float32)
                    bo = plsc.bitcast(w16 & jnp.int32(-65536), jnp.float32)
                    rowf[sb, r, pl.ds(32 * k, 16)] = ae * exs
                    rowf[sb, r, pl.ds(32 * k + 16, 16)] = bo * exs

    def step(j, b):
        bi = j % _KI
        sb = j % _SBUF
        wait_g(j, b, bi)
        compute(j, b, sb)
        start_s(j, sb, bi)

        @pl.when(j >= 1)
        def _():
            wait_s(j - 1, (j - 1) % _SBUF, (j - 1) % _KI)

        @pl.when(j + 2 < _NCHUNK)
        def _():
            wait_i(j + 2, (j + 2) % _KI)
            start_g(j + 2, (b + 2) % _NBUF, (j + 2) % _KI)

        @pl.when(j + 4 < _NCHUNK)
        def _():
            start_i(j + 4, (j + 4) % _KI)

    for j in range(4):
        start_i(j, j)
    wait_i(0, 0)
    start_g(0, 0, 0)
    wait_i(1, 1)
    start_g(1, 1, 1)

    nmain = (_NCHUNK // _NBUF) * _NBUF

    @pl.loop(0, _NCHUNK // _NBUF)
    def _(t):
        for b in range(_NBUF):
            step(t * _NBUF + b, b)

    for j in range(nmain, _NCHUNK):
        step(j, j % _NBUF)
    wait_s(_NCHUNK - 1, (_NCHUNK - 1) % _SBUF, (_NCHUNK - 1) % _KI)
    plsc.subcore_barrier()

    @pl.loop(0, _RPT // _ZC)
    def _(i):
        row0 = s * _RPT + i * _ZC
        pltpu.sync_copy(acc_sh.at[pl.ds(row0, _ZC), :], zsrc)
        pltpu.sync_copy(zsrc, acc_out.at[c, pl.ds(row0, _ZC), :])

    pltpu.sync_copy(dacc_sh.at[pl.ds(s * _RPT, _RPT)], z1d)
    pltpu.sync_copy(z1d, dacc_out.at[c, pl.ds(s * _RPT, _RPT)])


_sc_edge = pl.kernel(
    _sc_body,
    out_type=[
        jax.ShapeDtypeStruct((_NC, _NPAD, _D), jnp.float32),
        jax.ShapeDtypeStruct((_NC, _NPAD), jnp.float32),
    ],
    mesh=plsc.VectorSubcoreMesh(core_axis_name="c", subcore_axis_name="s",
                                num_cores=_NC, num_subcores=_NS),
    compiler_params=pltpu.CompilerParams(needs_layout_passes=False,
                                         use_tc_tiling_on_sc=False),
    scratch_types=[
        pltpu.VMEM((_KI, _CH), jnp.int32),
        pltpu.VMEM((_KI, _CH), jnp.int32),
        pltpu.VMEM((_SBUF, _CH), jnp.float32),
        pltpu.VMEM((_NBUF, _CH), jnp.float32),
        pltpu.VMEM((_NBUF, _CH), jnp.float32),
        pltpu.VMEM((16,), jnp.float32),
        pltpu.VMEM((_NBUF, _CH, _D // 2), jnp.int32),
        pltpu.VMEM((_SBUF, _CH, _D), jnp.float32),
        pltpu.VMEM((_RPT,), jnp.float32),
        pltpu.VMEM_SHARED((_NPAD, _D), jnp.float32),
        pltpu.VMEM_SHARED((_NPAD,), jnp.float32),
        pltpu.SemaphoreType.DMA((_NBUF,)),
        pltpu.SemaphoreType.DMA((_SBUF,)),
        pltpu.SemaphoreType.DMA((_KI,)),
    ],
)


def kernel(x0, edge_index, W0, att_src0, att_dst0, b0,
           W1, att_src1, att_dst1, b1):
    src = edge_index[0].astype(jnp.int32)
    dst = edge_index[1].astype(jnp.int32)
    npad_e = _EPW_PAD - _EPW
    pad_nodes = (jnp.arange(_NW * npad_e, dtype=jnp.int32) % _N).reshape(
        _NW, npad_e)
    srcp = jnp.concatenate([src.reshape(_NW, _EPW), pad_nodes],
                           axis=1).reshape(_NW * _EPW_PAD)
    dstp = jnp.concatenate([dst.reshape(_NW, _EPW), pad_nodes],
                           axis=1).reshape(_NW * _EPW_PAD)

    pmat = jnp.asarray(_PMAT)

    def _as_i32_rows(hb):
        return lax.bitcast_convert_type(hb.reshape(_N, _D // 2, 2),
                                        jnp.int32)

    h, acs, acd, bnd = _tc_transform(x0, W0, pmat, att_src0, att_dst0)
    acc, dacc = _sc_edge(_as_i32_rows(h), srcp, dstp,
                         acs.reshape(_N), acd.reshape(_N),
                         bnd.reshape(128)[:16])
    dv = (dacc[0] + dacc[1] + 1e-16).reshape(_NPAD, 1)
    x1, h1, acs1, acd1, bnd1 = _tc_combine_transform(
        acc, dv, b0.reshape(1, _D), W1, pmat, att_src1, att_dst1)
    acc1, dacc1 = _sc_edge(_as_i32_rows(h1), srcp, dstp, acs1.reshape(_N),
                           acd1.reshape(_N), bnd1.reshape(128)[:16])
    dv1 = (dacc1[0] + dacc1[1] + 1e-16).reshape(_NPAD, 1)
    x2 = _tc_combine(acc1, dv1, b1.reshape(1, _D))
    return (x0, x1, x2)


# speedup vs baseline: 1.9869x; 1.9821x over previous
import functools

import jax
import jax.numpy as jnp
import numpy as np
from jax import lax
from jax.experimental import pallas as pl
from jax.experimental.pallas import tpu as pltpu
from jax.experimental.pallas import tpu_sc as plsc

_N = 10000
_D = 128
_E = 320000
_NPAD = 10240
_NC = 2
_NS = 16
_NW = _NC * _NS
_EPW = _E // _NW
_CH = 112
_NCHUNK = 92
_EPW_PAD = _NCHUNK * _CH
_NBUF = 3
_KI = 6
_ZC = 64
_RPT = _NPAD // _NS
_SBUF = 2
_SLOPE = 0.2


def _emit_transform(x, w_ref, as_ref, ad_ref, hb_ref, acs_ref,
                    acd_ref, bnd_ref):
    w = w_ref[...]
    h = jnp.dot(x, w, preferred_element_type=jnp.float32)
    hb_ref[...] = h
    acs = lax.dot_general(h, as_ref[...], (((1,), (1,)), ((), ())),
                          preferred_element_type=jnp.float32)
    acd = lax.dot_general(h, ad_ref[...], (((1,), (1,)), ((), ())),
                          preferred_element_type=jnp.float32)
    acs_ref[...] = acs
    acd_ref[...] = acd
    bm = jnp.max(acs) + jnp.max(acd)
    bnd_ref[...] = jnp.full((1, 128), jnp.where(bm > 0, bm, _SLOPE * bm),
                            jnp.float32)


def _tc_transform_body(x_ref, w_ref, as_ref, ad_ref,
                       hb_ref, acs_ref, acd_ref, bnd_ref):
    _emit_transform(x_ref[...], w_ref, as_ref, ad_ref,
                    hb_ref, acs_ref, acd_ref, bnd_ref)


_tc_transform = pl.pallas_call(
    _tc_transform_body,
    out_shape=[
        jax.ShapeDtypeStruct((_N, _D), jnp.float32),
        jax.ShapeDtypeStruct((_N, 1), jnp.float32),
        jax.ShapeDtypeStruct((_N, 1), jnp.float32),
        jax.ShapeDtypeStruct((1, 128), jnp.float32),
    ],
)


def _emit_combine(acc_ref, dv_ref, b_ref):
    o = (acc_ref[0, 0:_N, :] + acc_ref[1, 0:_N, :]) / dv_ref[0:_N, :]
    o = o + b_ref[...]
    return jnp.where(o > 0, o, jnp.exp(jnp.minimum(o, 0.0)) - 1.0)


def _tc_combine_transform_body(acc_ref, dv_ref, b_ref, w_ref, as_ref,
                               ad_ref, x_ref, hb_ref, acs_ref, acd_ref,
                               bnd_ref):
    x = _emit_combine(acc_ref, dv_ref, b_ref)
    x_ref[...] = x
    _emit_transform(x, w_ref, as_ref, ad_ref, hb_ref, acs_ref,
                    acd_ref, bnd_ref)


_tc_combine_transform = pl.pallas_call(
    _tc_combine_transform_body,
    out_shape=[
        jax.ShapeDtypeStruct((_N, _D), jnp.float32),
        jax.ShapeDtypeStruct((_N, _D), jnp.float32),
        jax.ShapeDtypeStruct((_N, 1), jnp.float32),
        jax.ShapeDtypeStruct((_N, 1), jnp.float32),
        jax.ShapeDtypeStruct((1, 128), jnp.float32),
    ],
)


def _tc_combine_body(acc_ref, dv_ref, b_ref, x_ref):
    x_ref[...] = _emit_combine(acc_ref, dv_ref, b_ref)


_tc_combine = pl.pallas_call(
    _tc_combine_body,
    out_shape=jax.ShapeDtypeStruct((_N, _D), jnp.float32),
)


def _sc_body(h_hbm, srcf_hbm, dstf_hbm, as_hbm, ad_hbm, bnd_hbm,
             acc_out, dacc_out,
             srcc, dstc, exv, asg, adg, btab, rowb, z1d,
             acc_sh, dacc_sh, gsem, ssem, isem):
    c = lax.axis_index("c")
    s = lax.axis_index("s")
    wid = s * _NC + c
    ebase = wid * _EPW
    zero16 = jnp.zeros((16,), jnp.float32)

    @pl.loop(0, _ZC)
    def _(i):
        for k in range(_D // 16):
            rowb[0, i, pl.ds(k * 16, 16)] = zero16

    @pl.loop(0, _RPT // 16)
    def _(i):
        z1d[pl.ds(i * 16, 16)] = zero16

    zsrc = rowb.at[0, pl.ds(0, _ZC), :]

    @pl.loop(0, _RPT // _ZC)
    def _(i):
        pltpu.sync_copy(zsrc, acc_sh.at[pl.ds(s * _RPT + i * _ZC, _ZC), :])

    pltpu.sync_copy(z1d, dacc_sh.at[pl.ds(s * _RPT, _RPT)])
    plsc.subcore_barrier()

    pltpu.sync_copy(bnd_hbm, btab)
    bound = btab[...][0]

    lane = lax.iota(jnp.int32, 16)

    def start_i(j, bi):
        off = jnp.minimum(ebase + j * _CH, _E - _CH)
        pltpu.async_copy(srcf_hbm.at[pl.ds(off, _CH)], srcc.at[bi],
                         isem.at[bi])
        pltpu.async_copy(dstf_hbm.at[pl.ds(off, _CH)], dstc.at[bi],
                         isem.at[bi])

    def wait_i(j, bi):
        off = jnp.minimum(ebase + j * _CH, _E - _CH)
        pltpu.make_async_copy(srcf_hbm.at[pl.ds(off, _CH)], srcc.at[bi],
                              isem.at[bi]).wait()
        pltpu.make_async_copy(dstf_hbm.at[pl.ds(off, _CH)], dstc.at[bi],
                              isem.at[bi]).wait()

    def start_g(j, b, bi):
        pltpu.async_copy(h_hbm.at[srcc.at[bi]], rowb.at[b], gsem.at[b])
        pltpu.async_copy(as_hbm.at[srcc.at[bi]], asg.at[b], gsem.at[b])
        pltpu.async_copy(ad_hbm.at[dstc.at[bi]], adg.at[b], gsem.at[b])

    def wait_g(j, b, bi):
        pltpu.make_async_copy(h_hbm.at[srcc.at[bi]], rowb.at[b],
                              gsem.at[b]).wait()
        pltpu.make_async_copy(as_hbm.at[srcc.at[bi]], asg.at[b],
                              gsem.at[b]).wait()
        pltpu.make_async_copy(ad_hbm.at[dstc.at[bi]], adg.at[b],
                              gsem.at[b]).wait()

    def start_s(j, b, bi):
        pltpu.async_copy(rowb.at[b], acc_sh.at[dstc.at[bi]], ssem.at[b],
                         add=True)
        pltpu.async_copy(exv.at[b], dacc_sh.at[dstc.at[bi]], ssem.at[b],
                         add=True)

    def wait_s(j, b, bi):
        pltpu.make_async_copy(rowb.at[b], acc_sh.at[dstc.at[bi]],
                              ssem.at[b]).wait()
        pltpu.make_async_copy(exv.at[b], dacc_sh.at[dstc.at[bi]],
                              ssem.at[b]).wait()

    def compute(j, b):
        base = j * _CH
        for q in range(_CH // 16):
            sl16 = pl.ds(q * 16, 16)
            av = asg[b, sl16] + adg[b, sl16]
            e = jnp.where(av > 0, av, _SLOPE * av)
            ex = jnp.exp(e - bound)
            pos = base + q * 16 + lane
            ex = jnp.where(pos < _EPW, ex, 0.0)
            exv[b, sl16] = ex

        @pl.loop(0, _CH // 16)
        def _(g):
            exw = exv[b, pl.ds(g * 16, 16)]
            for l in range(16):
                r = g * 16 + l
                exs = exw[l]
                for k in range(_D // 16):
                    sl = pl.ds(16 * k, 16)
                    rowb[b, r, sl] = rowb[b, r, sl] * exs

    def step(j, b):
        bi = j % _KI
        wait_g(j, b, bi)
        compute(j, b)
        start_s(j, b, bi)

        @pl.when(j >= 1)
        def _():
            wait_s(j - 1, (j - 1) % _NBUF, (j - 1) % _KI)

        @pl.when(j + 2 < _NCHUNK)
        def _():
            wait_i(j + 2, (j + 2) % _KI)
            start_g(j + 2, (b + 2) % _NBUF, (j + 2) % _KI)

        @pl.when(j + 4 < _NCHUNK)
        def _():
            start_i(j + 4, (j + 4) % _KI)

    for j in range(4):
        start_i(j, j)
    wait_i(0, 0)
    start_g(0, 0, 0)
    wait_i(1, 1)
    start_g(1, 1, 1)

    nmain = (_NCHUNK // _NBUF) * _NBUF

    @pl.loop(0, _NCHUNK // _NBUF)
    def _(t):
        for b in range(_NBUF):
            step(t * _NBUF + b, b)

    for j in range(nmain, _NCHUNK):
        step(j, j % _NBUF)
    wait_s(_NCHUNK - 1, (_NCHUNK - 1) % _NBUF, (_NCHUNK - 1) % _KI)
    plsc.subcore_barrier()

    @pl.loop(0, _RPT // _ZC)
    def _(i):
        row0 = s * _RPT + i * _ZC
        pltpu.sync_copy(acc_sh.at[pl.ds(row0, _ZC), :], zsrc)
        pltpu.sync_copy(zsrc, acc_out.at[c, pl.ds(row0, _ZC), :])

    pltpu.sync_copy(dacc_sh.at[pl.ds(s * _RPT, _RPT)], z1d)
    pltpu.sync_copy(z1d, dacc_out.at[c, pl.ds(s * _RPT, _RPT)])


_sc_edge = pl.kernel(
    _sc_body,
    out_type=[
        jax.ShapeDtypeStruct((_NC, _NPAD, _D), jnp.float32),
        jax.ShapeDtypeStruct((_NC, _NPAD), jnp.float32),
    ],
    mesh=plsc.VectorSubcoreMesh(core_axis_name="c", subcore_axis_name="s",
                                num_cores=_NC, num_subcores=_NS),
    compiler_params=pltpu.CompilerParams(needs_layout_passes=False),
    scratch_types=[
        pltpu.VMEM((_KI, _CH), jnp.int32),
        pltpu.VMEM((_KI, _CH), jnp.int32),
        pltpu.VMEM((_NBUF, _CH), jnp.float32),
        pltpu.VMEM((_NBUF, _CH), jnp.float32),
        pltpu.VMEM((_NBUF, _CH), jnp.float32),
        pltpu.VMEM((16,), jnp.float32),
        pltpu.VMEM((_NBUF, _CH, _D), jnp.float32),
        pltpu.VMEM((_RPT,), jnp.float32),
        pltpu.VMEM_SHARED((_NPAD, _D), jnp.float32),
        pltpu.VMEM_SHARED((_NPAD,), jnp.float32),
        pltpu.SemaphoreType.DMA((_NBUF,)),
        pltpu.SemaphoreType.DMA((_NBUF,)),
        pltpu.SemaphoreType.DMA((_KI,)),
    ],
)


def kernel(x0, edge_index, W0, att_src0, att_dst0, b0,
           W1, att_src1, att_dst1, b1):
    edges = edge_index.astype(jnp.int32)
    srcf = edges[0]
    dstf = edges[1]
    h, acs, acd, bnd = _tc_transform(x0, W0, att_src0, att_dst0)
    acc, dacc = _sc_edge(h, srcf, dstf, acs.reshape(_N), acd.reshape(_N),
                         bnd.reshape(128)[:16])
    dv = (dacc[0] + dacc[1] + 1e-16).reshape(_NPAD, 1)
    x1, h1, acs1, acd1, bnd1 = _tc_combine_transform(
        acc, dv, b0.reshape(1, _D), W1, att_src1, att_dst1)
    acc1, dacc1 = _sc_edge(h1, srcf, dstf, acs1.reshape(_N),
                           acd1.reshape(_N), bnd1.reshape(128)[:16])
    dv1 = (dacc1[0] + dacc1[1] + 1e-16).reshape(_NPAD, 1)
    x2 = _tc_combine(acc1, dv1, b1.reshape(1, _D))
    return (x0, x1, x2)


# speedup vs baseline: 1.9885x; 1.0008x over previous
import functools

import jax
import jax.numpy as jnp
import numpy as np
from jax import lax
from jax.experimental import pallas as pl
from jax.experimental.pallas import tpu as pltpu
from jax.experimental.pallas import tpu_sc as plsc

_N = 10000
_D = 128
_E = 320000
_NPAD = 10240
_NC = 2
_NS = 16
_NW = _NC * _NS
_EPW = _E // _NW
_CH = 112
_NCHUNK = 92
_EPW_PAD = _NCHUNK * _CH
_NBUF = 3
_KI = 6
_ZC = 64
_RPT = _NPAD // _NS
_SBUF = 2
_SLOPE = 0.2


def _emit_transform(x, w_ref, as_ref, ad_ref, hb_ref, acs_ref,
                    acd_ref, bnd_ref):
    w = w_ref[...]
    h = jnp.dot(x, w, preferred_element_type=jnp.float32)
    hb_ref[...] = h
    acs = lax.dot_general(h, as_ref[...], (((1,), (1,)), ((), ())),
                          preferred_element_type=jnp.float32)
    acd = lax.dot_general(h, ad_ref[...], (((1,), (1,)), ((), ())),
                          preferred_element_type=jnp.float32)
    acs_ref[...] = acs
    acd_ref[...] = acd
    bm = jnp.max(acs) + jnp.max(acd)
    bnd_ref[...] = jnp.full((1, 128), jnp.where(bm > 0, bm, _SLOPE * bm),
                            jnp.float32)


def _tc_transform_body(x_ref, w_ref, as_ref, ad_ref,
                       hb_ref, acs_ref, acd_ref, bnd_ref):
    _emit_transform(x_ref[...], w_ref, as_ref, ad_ref,
                    hb_ref, acs_ref, acd_ref, bnd_ref)


_tc_transform = pl.pallas_call(
    _tc_transform_body,
    out_shape=[
        jax.ShapeDtypeStruct((_N, _D), jnp.float32),
        jax.ShapeDtypeStruct((_N, 1), jnp.float32),
        jax.ShapeDtypeStruct((_N, 1), jnp.float32),
        jax.ShapeDtypeStruct((1, 128), jnp.float32),
    ],
)


def _emit_combine(acc_ref, dv_ref, b_ref):
    o = (acc_ref[0, 0:_N, :] + acc_ref[1, 0:_N, :]) / dv_ref[0:_N, :]
    o = o + b_ref[...]
    return jnp.where(o > 0, o, jnp.exp(jnp.minimum(o, 0.0)) - 1.0)


def _tc_combine_transform_body(acc_ref, dv_ref, b_ref, w_ref, as_ref,
                               ad_ref, x_ref, hb_ref, acs_ref, acd_ref,
                               bnd_ref):
    x = _emit_combine(acc_ref, dv_ref, b_ref)
    x_ref[...] = x
    _emit_transform(x, w_ref, as_ref, ad_ref, hb_ref, acs_ref,
                    acd_ref, bnd_ref)


_tc_combine_transform = pl.pallas_call(
    _tc_combine_transform_body,
    out_shape=[
        jax.ShapeDtypeStruct((_N, _D), jnp.float32),
        jax.ShapeDtypeStruct((_N, _D), jnp.float32),
        jax.ShapeDtypeStruct((_N, 1), jnp.float32),
        jax.ShapeDtypeStruct((_N, 1), jnp.float32),
        jax.ShapeDtypeStruct((1, 128), jnp.float32),
    ],
)


def _tc_combine_body(acc_ref, dv_ref, b_ref, x_ref):
    x_ref[...] = _emit_combine(acc_ref, dv_ref, b_ref)


_tc_combine = pl.pallas_call(
    _tc_combine_body,
    out_shape=jax.ShapeDtypeStruct((_N, _D), jnp.float32),
)


def _sc_body(h_hbm, srcf_hbm, dstf_hbm, as_hbm, ad_hbm, bnd_hbm,
             acc_out, dacc_out,
             srcc, dstc, exv, asg, adg, btab, rowb, z1d,
             acc_sh, dacc_sh, gsem, ssem, isem):
    c = lax.axis_index("c")
    s = lax.axis_index("s")
    wid = s * _NC + c
    ebase = wid * _EPW
    zero16 = jnp.zeros((16,), jnp.float32)

    @pl.loop(0, _ZC)
    def _(i):
        for k in range(_D // 16):
            rowb[0, i, pl.ds(k * 16, 16)] = zero16

    @pl.loop(0, _RPT // 16)
    def _(i):
        z1d[pl.ds(i * 16, 16)] = zero16

    zsrc = rowb.at[0, pl.ds(0, _ZC), :]

    @pl.loop(0, _RPT // _ZC)
    def _(i):
        pltpu.sync_copy(zsrc, acc_sh.at[pl.ds(s * _RPT + i * _ZC, _ZC), :])

    pltpu.sync_copy(z1d, dacc_sh.at[pl.ds(s * _RPT, _RPT)])
    plsc.subcore_barrier()

    pltpu.sync_copy(bnd_hbm, btab)
    bound = btab[...][0]

    lane = lax.iota(jnp.int32, 16)

    def start_i(j, bi):
        off = jnp.minimum(ebase + j * _CH, _E - _CH)
        pltpu.async_copy(srcf_hbm.at[pl.ds(off, _CH)], srcc.at[bi],
                         isem.at[bi])
        pltpu.async_copy(dstf_hbm.at[pl.ds(off, _CH)], dstc.at[bi],
                         isem.at[bi])

    def wait_i(j, bi):
        off = jnp.minimum(ebase + j * _CH, _E - _CH)
        pltpu.make_async_copy(srcf_hbm.at[pl.ds(off, _CH)], srcc.at[bi],
                              isem.at[bi]).wait()
        pltpu.make_async_copy(dstf_hbm.at[pl.ds(off, _CH)], dstc.at[bi],
                              isem.at[bi]).wait()

    def start_g(j, b, bi):
        pltpu.async_copy(h_hbm.at[srcc.at[bi]], rowb.at[b], gsem.at[b])
        pltpu.async_copy(as_hbm.at[srcc.at[bi]], asg.at[b], gsem.at[b])
        pltpu.async_copy(ad_hbm.at[dstc.at[bi]], adg.at[b], gsem.at[b])

    def wait_g(j, b, bi):
        pltpu.make_async_copy(h_hbm.at[srcc.at[bi]], rowb.at[b],
                              gsem.at[b]).wait()
        pltpu.make_async_copy(as_hbm.at[srcc.at[bi]], asg.at[b],
                              gsem.at[b]).wait()
        pltpu.make_async_copy(ad_hbm.at[dstc.at[bi]], adg.at[b],
                              gsem.at[b]).wait()

    def start_s(j, b, bi):
        pltpu.async_copy(rowb.at[b], acc_sh.at[dstc.at[bi]], ssem.at[b],
                         add=True)
        pltpu.async_copy(exv.at[b], dacc_sh.at[dstc.at[bi]], ssem.at[b],
                         add=True)

    def wait_s(j, b, bi):
        pltpu.make_async_copy(rowb.at[b], acc_sh.at[dstc.at[bi]],
                              ssem.at[b]).wait()
        pltpu.make_async_copy(exv.at[b], dacc_sh.at[dstc.at[bi]],
                              ssem.at[b]).wait()

    def compute(j, b):
        base = j * _CH
        delta = jnp.maximum(0, ebase + base - (_E - _CH))
        for q in range(_CH // 16):
            sl16 = pl.ds(q * 16, 16)
            av = asg[b, sl16] + adg[b, sl16]
            e = jnp.where(av > 0, av, _SLOPE * av)
            ex = jnp.exp(e - bound)
            p = q * 16 + lane
            ex = jnp.where((p >= delta) & (base + p - delta < _EPW), ex, 0.0)
            exv[b, sl16] = ex

        @pl.loop(0, _CH // 16)
        def _(g):
            exw = exv[b, pl.ds(g * 16, 16)]
            for l in range(16):
                r = g * 16 + l
                exs = exw[l]
                for k in range(_D // 16):
                    sl = pl.ds(16 * k, 16)
                    rowb[b, r, sl] = rowb[b, r, sl] * exs

    def step(j, b):
        bi = j % _KI
        wait_g(j, b, bi)
        compute(j, b)
        start_s(j, b, bi)

        @pl.when(j >= 1)
        def _():
            wait_s(j - 1, (j - 1) % _NBUF, (j - 1) % _KI)

        @pl.when(j + 2 < _NCHUNK)
        def _():
            wait_i(j + 2, (j + 2) % _KI)
            start_g(j + 2, (b + 2) % _NBUF, (j + 2) % _KI)

        @pl.when(j + 4 < _NCHUNK)
        def _():
            start_i(j + 4, (j + 4) % _KI)

    for j in range(4):
        start_i(j, j)
    wait_i(0, 0)
    start_g(0, 0, 0)
    wait_i(1, 1)
    start_g(1, 1, 1)

    nmain = (_NCHUNK // _NBUF) * _NBUF

    @pl.loop(0, _NCHUNK // _NBUF)
    def _(t):
        for b in range(_NBUF):
            step(t * _NBUF + b, b)

    for j in range(nmain, _NCHUNK):
        step(j, j % _NBUF)
    wait_s(_NCHUNK - 1, (_NCHUNK - 1) % _NBUF, (_NCHUNK - 1) % _KI)
    plsc.subcore_barrier()

    @pl.loop(0, _RPT // _ZC)
    def _(i):
        row0 = s * _RPT + i * _ZC
        pltpu.sync_copy(acc_sh.at[pl.ds(row0, _ZC), :], zsrc)
        pltpu.sync_copy(zsrc, acc_out.at[c, pl.ds(row0, _ZC), :])

    pltpu.sync_copy(dacc_sh.at[pl.ds(s * _RPT, _RPT)], z1d)
    pltpu.sync_copy(z1d, dacc_out.at[c, pl.ds(s * _RPT, _RPT)])


_sc_edge = pl.kernel(
    _sc_body,
    out_type=[
        jax.ShapeDtypeStruct((_NC, _NPAD, _D), jnp.float32),
        jax.ShapeDtypeStruct((_NC, _NPAD), jnp.float32),
    ],
    mesh=plsc.VectorSubcoreMesh(core_axis_name="c", subcore_axis_name="s",
                                num_cores=_NC, num_subcores=_NS),
    compiler_params=pltpu.CompilerParams(needs_layout_passes=False),
    scratch_types=[
        pltpu.VMEM((_KI, _CH), jnp.int32),
        pltpu.VMEM((_KI, _CH), jnp.int32),
        pltpu.VMEM((_NBUF, _CH), jnp.float32),
        pltpu.VMEM((_NBUF, _CH), jnp.float32),
        pltpu.VMEM((_NBUF, _CH), jnp.float32),
        pltpu.VMEM((16,), jnp.float32),
        pltpu.VMEM((_NBUF, _CH, _D), jnp.float32),
        pltpu.VMEM((_RPT,), jnp.float32),
        pltpu.VMEM_SHARED((_NPAD, _D), jnp.float32),
        pltpu.VMEM_SHARED((_NPAD,), jnp.float32),
        pltpu.SemaphoreType.DMA((_NBUF,)),
        pltpu.SemaphoreType.DMA((_NBUF,)),
        pltpu.SemaphoreType.DMA((_KI,)),
    ],
)


def kernel(x0, edge_index, W0, att_src0, att_dst0, b0,
           W1, att_src1, att_dst1, b1):
    edges = edge_index.astype(jnp.int32)
    srcf = edges[0]
    dstf = edges[1]
    h, acs, acd, bnd = _tc_transform(x0, W0, att_src0, att_dst0)
    acc, dacc = _sc_edge(h, srcf, dstf, acs.reshape(_N), acd.reshape(_N),
                         bnd.reshape(128)[:16])
    dv = (dacc[0] + dacc[1] + 1e-16).reshape(_NPAD, 1)
    x1, h1, acs1, acd1, bnd1 = _tc_combine_transform(
        acc, dv, b0.reshape(1, _D), W1, att_src1, att_dst1)
    acc1, dacc1 = _sc_edge(h1, srcf, dstf, acs1.reshape(_N),
                           acd1.reshape(_N), bnd1.reshape(128)[:16])
    dv1 = (dacc1[0] + dacc1[1] + 1e-16).reshape(_NPAD, 1)
    x2 = _tc_combine(acc1, dv1, b1.reshape(1, _D))
    return (x0, x1, x2)
